# Initial kernel scaffold; baseline (speedup 1.0000x reference)
import functools

import jax
import jax.numpy as jnp
from jax import lax
from jax.experimental import pallas as pl
from jax.experimental.pallas import tpu as pltpu
from jax.experimental.pallas import tpu_sc as plsc

N = 10000
E = 320000
H = 128
NG = 256
L = 4
V = 119
NP = 10240
BT = 1024
GRID = NP // BT
NC = 2
NS = 16
NW = NC * NS
CH = 128
NCH = 80
EPW = NCH * CH
E_PAD = NW * EPW
RPT = NP // NS
DEGW = 16

_f32 = jnp.float32


def _sc_deg_call(dst_r, ones16, zeros16):
    mesh = plsc.VectorSubcoreMesh(core_axis_name="c", subcore_axis_name="s")

    @functools.partial(
        pl.kernel,
        mesh=mesh,
        out_type=jax.ShapeDtypeStruct((NC, NP, DEGW), _f32),
        scratch_types=[
            pltpu.VMEM((NCH, CH), jnp.int32),
            pltpu.VMEM((CH, DEGW), _f32),
            pltpu.VMEM_SHARED((NP, DEGW), _f32),
        ],
    )
    def k(dst_hbm, ones_hbm, zeros_hbm, out_hbm, dst_v, ones_v, acc_sh):
        cid = lax.axis_index("c")
        sid = lax.axis_index("s")
        wid = cid * NS + sid
        pltpu.sync_copy(zeros_hbm, acc_sh.at[pl.ds(sid * RPT, RPT)])
        pltpu.sync_copy(ones_hbm, ones_v)
        pltpu.sync_copy(dst_hbm.at[wid], dst_v)
        plsc.subcore_barrier()

        def body(j, carry):
            pltpu.sync_copy(ones_v, acc_sh.at[dst_v.at[j]], add=True)
            return carry

        lax.fori_loop(0, NCH, body, 0)
        plsc.subcore_barrier()
        pltpu.sync_copy(acc_sh.at[pl.ds(sid * RPT, RPT)],
                        out_hbm.at[cid, pl.ds(sid * RPT, RPT)])

    return k(dst_r, ones16, zeros16)


def _sc_scatter_call(ms, src_r, dst_r, zeros_h):
    mesh = plsc.VectorSubcoreMesh(core_axis_name="c", subcore_axis_name="s")

    @functools.partial(
        pl.kernel,
        mesh=mesh,
        out_type=jax.ShapeDtypeStruct((NC, NP, H), _f32),
        scratch_types=[
            pltpu.VMEM((NCH, CH), jnp.int32),
            pltpu.VMEM((NCH, CH), jnp.int32),
            pltpu.VMEM((CH, H), _f32),
            pltpu.VMEM_SHARED((NP, H), _f32),
            pltpu.SemaphoreType.DMA,
        ],
    )
    def k(ms_hbm, src_hbm, dst_hbm, zeros_hbm, out_hbm,
          src_v, dst_v, rows_v, acc_sh, sem):
        cid = lax.axis_index("c")
        sid = lax.axis_index("s")
        wid = cid * NS + sid
        pltpu.sync_copy(zeros_hbm, acc_sh.at[pl.ds(sid * RPT, RPT)])
        pltpu.sync_copy(src_hbm.at[wid], src_v)
        pltpu.sync_copy(dst_hbm.at[wid], dst_v)
        plsc.subcore_barrier()

        def body(j, carry):
            pltpu.async_copy(ms_hbm.at[src_v.at[j]], rows_v, sem).wait()
            pltpu.sync_copy(rows_v, acc_sh.at[dst_v.at[j]], add=True)
            return carry

        lax.fori_loop(0, NCH, body, 0)
        plsc.subcore_barrier()
        pltpu.sync_copy(acc_sh.at[pl.ds(sid * RPT, RPT)],
                        out_hbm.at[cid, pl.ds(sid * RPT, RPT)])

    return k(ms, src_r, dst_r, zeros_h)


def _dot(a, b):
    return jnp.dot(a, b, preferred_element_type=_f32)


def _encoder_body(x_ref, d0_ref, d1_ref, emb_ref, w0_ref,
                  dis_ref, h_ref, ms_ref):
    xb = x_ref[0, 0, :]
    oh = (xb[:, None] == lax.broadcasted_iota(jnp.int32, (BT, H), 1)).astype(_f32)
    h = _dot(oh, emb_ref[...])
    deg = 1.0 + d0_ref[:, 0:1] + d1_ref[:, 0:1]
    dis = lax.rsqrt(jnp.maximum(deg, 1.0))
    disb = jnp.broadcast_to(dis, (BT, H))
    dis_ref[...] = disb
    h_ref[...] = h
    ms_ref[...] = disb * _dot(h, w0_ref[...])


def _tc_encoder(x_r, d0, d1, emb_p, w0):
    return pl.pallas_call(
        _encoder_body,
        grid=(GRID,),
        in_specs=[
            pl.BlockSpec((1, 1, BT), lambda n: (n, 0, 0)),
            pl.BlockSpec((BT, DEGW), lambda n: (n, 0)),
            pl.BlockSpec((BT, DEGW), lambda n: (n, 0)),
            pl.BlockSpec((H, H), lambda n: (0, 0)),
            pl.BlockSpec((H, H), lambda n: (0, 0)),
        ],
        out_specs=[
            pl.BlockSpec((BT, H), lambda n: (n, 0)),
            pl.BlockSpec((BT, H), lambda n: (n, 0)),
            pl.BlockSpec((BT, H), lambda n: (n, 0)),
        ],
        out_shape=[
            jax.ShapeDtypeStruct((NP, H), _f32),
            jax.ShapeDtypeStruct((NP, H), _f32),
            jax.ShapeDtypeStruct((NP, H), _f32),
        ],
    )(x_r, d0, d1, emb_p, w0)


def _layer_math(p0, p1, ms, h, dis, bc_r, g_r, b_r):
    t = p0 + p1 + ms
    agg = dis * t + bc_r
    hh = jax.nn.gelu(agg) + h
    mu = jnp.mean(hh, axis=1, keepdims=True)
    var = jnp.mean((hh - mu) ** 2, axis=1, keepdims=True)
    return (hh - mu) * lax.rsqrt(var + 1e-5) * g_r + b_r


def _layer_body(p0_ref, p1_ref, ms_ref, h_ref, dis_ref, w_ref,
                bc_ref, g_ref, b_ref, hn_ref, msn_ref):
    hn = _layer_math(p0_ref[...], p1_ref[...], ms_ref[...], h_ref[...],
                     dis_ref[...], bc_ref[...], g_ref[...], b_ref[...])
    hn_ref[...] = hn
    msn_ref[...] = dis_ref[...] * _dot(hn, w_ref[...])


def _tc_layer(p0, p1, ms, h, dis, w_next, bc_r, g_r, b_r):
    blk = pl.BlockSpec((BT, H), lambda n: (n, 0))
    row = pl.BlockSpec((1, H), lambda n: (0, 0))
    return pl.pallas_call(
        _layer_body,
        grid=(GRID,),
        in_specs=[blk, blk, blk, blk, blk,
                  pl.BlockSpec((H, H), lambda n: (0, 0)), row, row, row],
        out_specs=[blk, blk],
        out_shape=[
            jax.ShapeDtypeStruct((NP, H), _f32),
            jax.ShapeDtypeStruct((NP, H), _f32),
        ],
    )(p0, p1, ms, h, dis, w_next, bc_r, g_r, b_r)


def _final_body(p0_ref, p1_ref, ms_ref, h_ref, dis_ref, batch_ref,
                bc_ref, g_ref, b_ref, w1_ref, b1_ref, bng_ref, bnb_ref,
                w2_ref, b2_ref, out_ref, acc_pool, acc_cnt):
    n = pl.program_id(0)

    @pl.when(n == 0)
    def _():
        acc_pool[...] = jnp.zeros((NG, H), _f32)
        acc_cnt[...] = jnp.zeros((NG, H), _f32)

    hn = _layer_math(p0_ref[...], p1_ref[...], ms_ref[...], h_ref[...],
                     dis_ref[...], bc_ref[...], g_ref[...], b_ref[...])
    bb = batch_ref[0, 0, :]
    oh = (bb[None, :] == lax.broadcasted_iota(jnp.int32, (NG, BT), 0)).astype(_f32)
    acc_pool[...] = acc_pool[...] + _dot(oh, hn)
    cnt = jnp.sum(oh, axis=1, keepdims=True)
    acc_cnt[...] = acc_cnt[...] + jnp.broadcast_to(cnt, (NG, H))

    @pl.when(n == GRID - 1)
    def _():
        pooled = acc_pool[...] / jnp.maximum(acc_cnt[...], 1.0)
        z = _dot(pooled, w1_ref[...]) + b1_ref[...]
        z = z * bng_ref[...] * lax.rsqrt(jnp.float32(1.0 + 1e-5)) + bnb_ref[...]
        z = jax.nn.gelu(z)
        out_ref[...] = _dot(z, w2_ref[...]) + b2_ref[...]


def _tc_final(p0, p1, ms, h, dis, batch_r, bc_r, g_r, b_r,
              w1, b1_r, bng_r, bnb_r, w2, b2_r):
    blk = pl.BlockSpec((BT, H), lambda n: (n, 0))
    row = pl.BlockSpec((1, H), lambda n: (0, 0))
    mat = pl.BlockSpec((H, H), lambda n: (0, 0))
    return pl.pallas_call(
        _final_body,
        grid=(GRID,),
        in_specs=[blk, blk, blk, blk, blk,
                  pl.BlockSpec((1, 1, BT), lambda n: (n, 0, 0)),
                  row, row, row, mat, row, row, row, mat, row],
        out_specs=pl.BlockSpec((NG, H), lambda n: (0, 0)),
        out_shape=jax.ShapeDtypeStruct((NG, H), _f32),
        scratch_shapes=[
            pltpu.VMEM((NG, H), _f32),
            pltpu.VMEM((NG, H), _f32),
        ],
    )(p0, p1, ms, h, dis, batch_r, bc_r, g_r, b_r,
      w1, b1_r, bng_r, bnb_r, w2, b2_r)


def kernel(x, edge_index, edge_attr, batch, emb, Wc, bc, ln_g, ln_b,
           W1, b1, bn_g, bn_b, W2, b2):
    del edge_attr
    x = x.astype(jnp.int32)
    ei = edge_index.astype(jnp.int32)
    batch = batch.astype(jnp.int32)

    pad_e = E_PAD - E
    dummy_dst = N + (jnp.arange(pad_e, dtype=jnp.int32) % (NP - N))
    src_r = jnp.concatenate(
        [ei[0], jnp.zeros((pad_e,), jnp.int32)]).reshape(NW, NCH, CH)
    dst_r = jnp.concatenate([ei[1], dummy_dst]).reshape(NW, NCH, CH)

    x_r = jnp.pad(x, (0, NP - N), constant_values=V).reshape(GRID, 1, BT)
    batch_r = jnp.pad(batch, (0, NP - N),
                      constant_values=NG).reshape(GRID, 1, BT)
    emb_p = jnp.pad(emb, ((0, H - V), (0, 0)))

    ones16 = jnp.ones((CH, DEGW), _f32)
    zeros16 = jnp.zeros((RPT, DEGW), _f32)
    zeros_h = jnp.zeros((RPT, H), _f32)

    degp = _sc_deg_call(dst_r, ones16, zeros16)
    dis, h, ms = _tc_encoder(x_r, degp[0], degp[1], emb_p, Wc[0])

    for l in range(L - 1):
        part = _sc_scatter_call(ms, src_r, dst_r, zeros_h)
        h, ms = _tc_layer(part[0], part[1], ms, h, dis, Wc[l + 1],
                          bc[l].reshape(1, H), ln_g[l].reshape(1, H),
                          ln_b[l].reshape(1, H))

    part = _sc_scatter_call(ms, src_r, dst_r, zeros_h)
    out = _tc_final(part[0], part[1], ms, h, dis, batch_r,
                    bc[L - 1].reshape(1, H), ln_g[L - 1].reshape(1, H),
                    ln_b[L - 1].reshape(1, H),
                    W1, b1.reshape(1, H), bn_g.reshape(1, H),
                    bn_b.reshape(1, H), W2, b2.reshape(1, H))
    return out


# speedup vs baseline: 6.0089x; 6.0089x over previous
import functools

import jax
import jax.numpy as jnp
from jax import lax
from jax.experimental import pallas as pl
from jax.experimental.pallas import tpu as pltpu
from jax.experimental.pallas import tpu_sc as plsc

N = 10000
E = 320000
H = 128
NG = 256
L = 4
V = 119
NP = 10240
BT = 1024
GRID = NP // BT
NC = 2
NS = 16
NW = NC * NS
CH = 128
NCH = 80
EPW = NCH * CH
E_PAD = NW * EPW
RPT = NP // NS
DEGW = 16

_f32 = jnp.float32


def _sc_deg_call(dst_r, ones_h, zeros_h):
    mesh = plsc.VectorSubcoreMesh(core_axis_name="c", subcore_axis_name="s")

    @functools.partial(
        pl.kernel,
        mesh=mesh,
        out_type=jax.ShapeDtypeStruct((NC, NP, H), _f32),
        scratch_types=[
            pltpu.VMEM((NCH, CH), jnp.int32),
            pltpu.VMEM((CH, H), _f32),
            pltpu.VMEM_SHARED((NP, H), _f32),
        ],
    )
    def k(dst_hbm, ones_hbm, zeros_hbm, out_hbm, dst_v, ones_v, acc_sh):
        cid = lax.axis_index("c")
        sid = lax.axis_index("s")
        wid = cid * NS + sid
        pltpu.sync_copy(zeros_hbm, acc_sh.at[pl.ds(sid * RPT, RPT)])
        pltpu.sync_copy(ones_hbm, ones_v)
        pltpu.sync_copy(dst_hbm.at[wid], dst_v)
        plsc.subcore_barrier()

        def body(j, carry):
            pltpu.sync_copy(ones_v, acc_sh.at[dst_v.at[j]], add=True)
            return carry

        lax.fori_loop(0, NCH, body, 0)
        plsc.subcore_barrier()
        pltpu.sync_copy(acc_sh.at[pl.ds(sid * RPT, RPT)],
                        out_hbm.at[cid, pl.ds(sid * RPT, RPT)])

    return k(dst_r, ones_h, zeros_h)


def _sc_scatter_call(ms, src_r, dst_r, zeros_h):
    mesh = plsc.VectorSubcoreMesh(core_axis_name="c", subcore_axis_name="s")

    @functools.partial(
        pl.kernel,
        mesh=mesh,
        out_type=jax.ShapeDtypeStruct((NC, NP, H), _f32),
        scratch_types=[
            pltpu.VMEM((NCH, CH), jnp.int32),
            pltpu.VMEM((NCH, CH), jnp.int32),
            pltpu.VMEM((CH, H), _f32),
            pltpu.VMEM_SHARED((NP, H), _f32),
            pltpu.SemaphoreType.DMA,
        ],
    )
    def k(ms_hbm, src_hbm, dst_hbm, zeros_hbm, out_hbm,
          src_v, dst_v, rows_v, acc_sh, sem):
        cid = lax.axis_index("c")
        sid = lax.axis_index("s")
        wid = cid * NS + sid
        pltpu.sync_copy(zeros_hbm, acc_sh.at[pl.ds(sid * RPT, RPT)])
        pltpu.sync_copy(src_hbm.at[wid], src_v)
        pltpu.sync_copy(dst_hbm.at[wid], dst_v)
        plsc.subcore_barrier()

        def body(j, carry):
            pltpu.async_copy(ms_hbm.at[src_v.at[j]], rows_v, sem).wait()
            pltpu.sync_copy(rows_v, acc_sh.at[dst_v.at[j]], add=True)
            return carry

        lax.fori_loop(0, NCH, body, 0)
        plsc.subcore_barrier()
        pltpu.sync_copy(acc_sh.at[pl.ds(sid * RPT, RPT)],
                        out_hbm.at[cid, pl.ds(sid * RPT, RPT)])

    return k(ms, src_r, dst_r, zeros_h)


def _dot(a, b):
    return jnp.dot(a, b, preferred_element_type=_f32)


def _encoder_body(x_ref, d0_ref, d1_ref, emb_ref, w0_ref,
                  dis_ref, h_ref, ms_ref):
    xb = x_ref[0, 0, :]
    oh = (xb[:, None] == lax.broadcasted_iota(jnp.int32, (BT, H), 1)).astype(_f32)
    h = _dot(oh, emb_ref[...])
    deg = 1.0 + d0_ref[:, 0:1] + d1_ref[:, 0:1]
    dis = lax.rsqrt(jnp.maximum(deg, 1.0))
    disb = jnp.broadcast_to(dis, (BT, H))
    dis_ref[...] = disb
    h_ref[...] = h
    ms_ref[...] = disb * _dot(h, w0_ref[...])


def _tc_encoder(x_r, d0, d1, emb_p, w0):
    return pl.pallas_call(
        _encoder_body,
        grid=(GRID,),
        in_specs=[
            pl.BlockSpec((1, 1, BT), lambda n: (n, 0, 0)),
            pl.BlockSpec((BT, H), lambda n: (n, 0)),
            pl.BlockSpec((BT, H), lambda n: (n, 0)),
            pl.BlockSpec((H, H), lambda n: (0, 0)),
            pl.BlockSpec((H, H), lambda n: (0, 0)),
        ],
        out_specs=[
            pl.BlockSpec((BT, H), lambda n: (n, 0)),
            pl.BlockSpec((BT, H), lambda n: (n, 0)),
            pl.BlockSpec((BT, H), lambda n: (n, 0)),
        ],
        out_shape=[
            jax.ShapeDtypeStruct((NP, H), _f32),
            jax.ShapeDtypeStruct((NP, H), _f32),
            jax.ShapeDtypeStruct((NP, H), _f32),
        ],
    )(x_r, d0, d1, emb_p, w0)


def _layer_math(p0, p1, ms, h, dis, bc_r, g_r, b_r):
    t = p0 + p1 + ms
    agg = dis * t + bc_r
    hh = jax.nn.gelu(agg) + h
    mu = jnp.mean(hh, axis=1, keepdims=True)
    var = jnp.mean((hh - mu) ** 2, axis=1, keepdims=True)
    return (hh - mu) * lax.rsqrt(var + 1e-5) * g_r + b_r


def _layer_body(p0_ref, p1_ref, ms_ref, h_ref, dis_ref, w_ref,
                bc_ref, g_ref, b_ref, hn_ref, msn_ref):
    hn = _layer_math(p0_ref[...], p1_ref[...], ms_ref[...], h_ref[...],
                     dis_ref[...], bc_ref[...], g_ref[...], b_ref[...])
    hn_ref[...] = hn
    msn_ref[...] = dis_ref[...] * _dot(hn, w_ref[...])


def _tc_layer(p0, p1, ms, h, dis, w_next, bc_r, g_r, b_r):
    blk = pl.BlockSpec((BT, H), lambda n: (n, 0))
    row = pl.BlockSpec((1, H), lambda n: (0, 0))
    return pl.pallas_call(
        _layer_body,
        grid=(GRID,),
        in_specs=[blk, blk, blk, blk, blk,
                  pl.BlockSpec((H, H), lambda n: (0, 0)), row, row, row],
        out_specs=[blk, blk],
        out_shape=[
            jax.ShapeDtypeStruct((NP, H), _f32),
            jax.ShapeDtypeStruct((NP, H), _f32),
        ],
    )(p0, p1, ms, h, dis, w_next, bc_r, g_r, b_r)


def _final_body(p0_ref, p1_ref, ms_ref, h_ref, dis_ref, batch_ref,
                bc_ref, g_ref, b_ref, w1_ref, b1_ref, bng_ref, bnb_ref,
                w2_ref, b2_ref, out_ref, acc_pool, acc_cnt):
    n = pl.program_id(0)

    @pl.when(n == 0)
    def _():
        acc_pool[...] = jnp.zeros((NG, H), _f32)
        acc_cnt[...] = jnp.zeros((NG, H), _f32)

    hn = _layer_math(p0_ref[...], p1_ref[...], ms_ref[...], h_ref[...],
                     dis_ref[...], bc_ref[...], g_ref[...], b_ref[...])
    bb = batch_ref[0, 0, :]
    oh = (bb[None, :] == lax.broadcasted_iota(jnp.int32, (NG, BT), 0)).astype(_f32)
    acc_pool[...] = acc_pool[...] + _dot(oh, hn)
    cnt = jnp.sum(oh, axis=1, keepdims=True)
    acc_cnt[...] = acc_cnt[...] + jnp.broadcast_to(cnt, (NG, H))

    @pl.when(n == GRID - 1)
    def _():
        pooled = acc_pool[...] / jnp.maximum(acc_cnt[...], 1.0)
        z = _dot(pooled, w1_ref[...]) + b1_ref[...]
        z = z * bng_ref[...] * lax.rsqrt(jnp.float32(1.0 + 1e-5)) + bnb_ref[...]
        z = jax.nn.gelu(z)
        out_ref[...] = _dot(z, w2_ref[...]) + b2_ref[...]


def _tc_final(p0, p1, ms, h, dis, batch_r, bc_r, g_r, b_r,
              w1, b1_r, bng_r, bnb_r, w2, b2_r):
    blk = pl.BlockSpec((BT, H), lambda n: (n, 0))
    row = pl.BlockSpec((1, H), lambda n: (0, 0))
    mat = pl.BlockSpec((H, H), lambda n: (0, 0))
    return pl.pallas_call(
        _final_body,
        grid=(GRID,),
        in_specs=[blk, blk, blk, blk, blk,
                  pl.BlockSpec((1, 1, BT), lambda n: (n, 0, 0)),
                  row, row, row, mat, row, row, row, mat, row],
        out_specs=pl.BlockSpec((NG, H), lambda n: (0, 0)),
        out_shape=jax.ShapeDtypeStruct((NG, H), _f32),
        scratch_shapes=[
            pltpu.VMEM((NG, H), _f32),
            pltpu.VMEM((NG, H), _f32),
        ],
    )(p0, p1, ms, h, dis, batch_r, bc_r, g_r, b_r,
      w1, b1_r, bng_r, bnb_r, w2, b2_r)


def kernel(x, edge_index, edge_attr, batch, emb, Wc, bc, ln_g, ln_b,
           W1, b1, bn_g, bn_b, W2, b2):
    del edge_attr
    x = x.astype(jnp.int32)
    ei = edge_index.astype(jnp.int32)
    batch = batch.astype(jnp.int32)

    pad_e = E_PAD - E
    dummy_dst = N + (jnp.arange(pad_e, dtype=jnp.int32) % (NP - N))
    src_r = jnp.concatenate(
        [ei[0], jnp.zeros((pad_e,), jnp.int32)]).reshape(NW, NCH, CH)
    dst_r = jnp.concatenate([ei[1], dummy_dst]).reshape(NW, NCH, CH)

    x_r = jnp.pad(x, (0, NP - N), constant_values=V).reshape(GRID, 1, BT)
    batch_r = jnp.pad(batch, (0, NP - N),
                      constant_values=NG).reshape(GRID, 1, BT)
    emb_p = jnp.pad(emb, ((0, H - V), (0, 0)))

    ones_h = jnp.ones((CH, H), _f32)
    zeros_h = jnp.zeros((RPT, H), _f32)

    degp = _sc_deg_call(dst_r, ones_h, zeros_h)
    dis, h, ms = _tc_encoder(x_r, degp[0], degp[1], emb_p, Wc[0])

    for l in range(L - 1):
        part = _sc_scatter_call(ms, src_r, dst_r, zeros_h)
        h, ms = _tc_layer(part[0], part[1], ms, h, dis, Wc[l + 1],
                          bc[l].reshape(1, H), ln_g[l].reshape(1, H),
                          ln_b[l].reshape(1, H))

    part = _sc_scatter_call(ms, src_r, dst_r, zeros_h)
    out = _tc_final(part[0], part[1], ms, h, dis, batch_r,
                    bc[L - 1].reshape(1, H), ln_g[L - 1].reshape(1, H),
                    ln_b[L - 1].reshape(1, H),
                    W1, b1.reshape(1, H), bn_g.reshape(1, H),
                    bn_b.reshape(1, H), W2, b2.reshape(1, H))
    return out


# speedup vs baseline: 6.4449x; 1.0726x over previous
import functools

import jax
import jax.numpy as jnp
from jax import lax
from jax.experimental import pallas as pl
from jax.experimental.pallas import tpu as pltpu
from jax.experimental.pallas import tpu_sc as plsc

N = 10000
E = 320000
H = 128
NG = 256
L = 4
V = 119
NP = 10240
BT = 1024
GRID = NP // BT
NC = 2
NS = 16
NW = NC * NS
CH = 128
NCH = 80
EPW = NCH * CH
E_PAD = NW * EPW
RPT = NP // NS
NBLK = 40

_f32 = jnp.float32


def _sc_deg_call(dst_r, ones_h, zeros_h):
    mesh = plsc.VectorSubcoreMesh(core_axis_name="c", subcore_axis_name="s")

    @functools.partial(
        pl.kernel,
        mesh=mesh,
        out_type=jax.ShapeDtypeStruct((NC, NP, H), _f32),
        scratch_types=[
            pltpu.VMEM((NCH, CH), jnp.int32),
            pltpu.VMEM((CH, H), _f32),
            pltpu.VMEM_SHARED((NP, H), _f32),
        ],
    )
    def k(dst_hbm, ones_hbm, zeros_hbm, out_hbm, dst_v, ones_v, acc_sh):
        cid = lax.axis_index("c")
        sid = lax.axis_index("s")
        wid = cid * NS + sid
        pltpu.sync_copy(zeros_hbm, acc_sh.at[pl.ds(sid * RPT, RPT)])
        pltpu.sync_copy(ones_hbm, ones_v)
        pltpu.sync_copy(dst_hbm.at[wid], dst_v)
        plsc.subcore_barrier()

        def body(j, carry):
            pltpu.sync_copy(ones_v, acc_sh.at[dst_v.at[j]], add=True)
            return carry

        lax.fori_loop(0, NCH, body, 0)
        plsc.subcore_barrier()
        pltpu.sync_copy(acc_sh.at[pl.ds(sid * RPT, RPT)],
                        out_hbm.at[cid, pl.ds(sid * RPT, RPT)])

    return k(dst_r, ones_h, zeros_h)


def _sc_scatter_call(ms, src_r, dst_r, zeros_h):
    mesh = plsc.VectorSubcoreMesh(core_axis_name="c", subcore_axis_name="s")

    @functools.partial(
        pl.kernel,
        mesh=mesh,
        out_type=jax.ShapeDtypeStruct((NC, NP, H), _f32),
        scratch_types=[
            pltpu.VMEM((NBLK, CH), jnp.int32),
            pltpu.VMEM((NBLK, CH), jnp.int32),
            pltpu.VMEM((CH, H), _f32),
            pltpu.VMEM((CH, H), _f32),
            pltpu.VMEM_SHARED((NP, H), _f32),
            pltpu.SemaphoreType.DMA,
            pltpu.SemaphoreType.DMA,
        ],
    )
    def k(ms_hbm, src_hbm, dst_hbm, zeros_hbm, out_hbm,
          src_v, dst_v, buf0, buf1, acc_sh, g0, g1):
        cid = lax.axis_index("c")
        sid = lax.axis_index("s")
        wid = cid * NS + sid
        pltpu.sync_copy(zeros_hbm, acc_sh.at[pl.ds(sid * RPT, RPT)])
        plsc.subcore_barrier()

        bufs = (buf0, buf1)
        sems = (g0, g1)
        for hb in range(NCH // NBLK):
            pltpu.sync_copy(src_hbm.at[wid, pl.ds(hb * NBLK, NBLK)], src_v)
            pltpu.sync_copy(dst_hbm.at[wid, pl.ds(hb * NBLK, NBLK)], dst_v)
            pltpu.async_copy(ms_hbm.at[src_v.at[0]], buf0, g0)

            def body(i, carry):
                for b in range(2):
                    j = 2 * i + b
                    pltpu.make_async_copy(
                        ms_hbm.at[src_v.at[j]], bufs[b], sems[b]).wait()

                    @pl.when(j + 1 < NBLK)
                    def _():
                        pltpu.async_copy(ms_hbm.at[src_v.at[j + 1]],
                                         bufs[1 - b], sems[1 - b])

                    pltpu.sync_copy(bufs[b], acc_sh.at[dst_v.at[j]],
                                    add=True)
                return carry

            lax.fori_loop(0, NBLK // 2, body, 0)

        plsc.subcore_barrier()
        pltpu.sync_copy(acc_sh.at[pl.ds(sid * RPT, RPT)],
                        out_hbm.at[cid, pl.ds(sid * RPT, RPT)])

    return k(ms, src_r, dst_r, zeros_h)


def _dot(a, b):
    return jnp.dot(a, b, preferred_element_type=_f32)


def _encoder_body(x_ref, d0_ref, d1_ref, emb_ref, w0_ref,
                  dis_ref, h_ref, ms_ref):
    xb = x_ref[0, 0, :]
    oh = (xb[:, None] == lax.broadcasted_iota(jnp.int32, (BT, H), 1)).astype(_f32)
    h = _dot(oh, emb_ref[...])
    deg = 1.0 + d0_ref[:, 0:1] + d1_ref[:, 0:1]
    dis = lax.rsqrt(jnp.maximum(deg, 1.0))
    disb = jnp.broadcast_to(dis, (BT, H))
    dis_ref[...] = disb
    h_ref[...] = h
    ms_ref[...] = disb * _dot(h, w0_ref[...])


def _tc_encoder(x_r, d0, d1, emb_p, w0):
    return pl.pallas_call(
        _encoder_body,
        grid=(GRID,),
        in_specs=[
            pl.BlockSpec((1, 1, BT), lambda n: (n, 0, 0)),
            pl.BlockSpec((BT, H), lambda n: (n, 0)),
            pl.BlockSpec((BT, H), lambda n: (n, 0)),
            pl.BlockSpec((H, H), lambda n: (0, 0)),
            pl.BlockSpec((H, H), lambda n: (0, 0)),
        ],
        out_specs=[
            pl.BlockSpec((BT, H), lambda n: (n, 0)),
            pl.BlockSpec((BT, H), lambda n: (n, 0)),
            pl.BlockSpec((BT, H), lambda n: (n, 0)),
        ],
        out_shape=[
            jax.ShapeDtypeStruct((NP, H), _f32),
            jax.ShapeDtypeStruct((NP, H), _f32),
            jax.ShapeDtypeStruct((NP, H), _f32),
        ],
    )(x_r, d0, d1, emb_p, w0)


def _layer_math(p0, p1, ms, h, dis, bc_r, g_r, b_r):
    t = p0 + p1 + ms
    agg = dis * t + bc_r
    hh = jax.nn.gelu(agg) + h
    mu = jnp.mean(hh, axis=1, keepdims=True)
    var = jnp.mean((hh - mu) ** 2, axis=1, keepdims=True)
    return (hh - mu) * lax.rsqrt(var + 1e-5) * g_r + b_r


def _layer_body(p0_ref, p1_ref, ms_ref, h_ref, dis_ref, w_ref,
                bc_ref, g_ref, b_ref, hn_ref, msn_ref):
    hn = _layer_math(p0_ref[...], p1_ref[...], ms_ref[...], h_ref[...],
                     dis_ref[...], bc_ref[...], g_ref[...], b_ref[...])
    hn_ref[...] = hn
    msn_ref[...] = dis_ref[...] * _dot(hn, w_ref[...])


def _tc_layer(p0, p1, ms, h, dis, w_next, bc_r, g_r, b_r):
    blk = pl.BlockSpec((BT, H), lambda n: (n, 0))
    row = pl.BlockSpec((1, H), lambda n: (0, 0))
    return pl.pallas_call(
        _layer_body,
        grid=(GRID,),
        in_specs=[blk, blk, blk, blk, blk,
                  pl.BlockSpec((H, H), lambda n: (0, 0)), row, row, row],
        out_specs=[blk, blk],
        out_shape=[
            jax.ShapeDtypeStruct((NP, H), _f32),
            jax.ShapeDtypeStruct((NP, H), _f32),
        ],
    )(p0, p1, ms, h, dis, w_next, bc_r, g_r, b_r)


def _final_body(p0_ref, p1_ref, ms_ref, h_ref, dis_ref, batch_ref,
                bc_ref, g_ref, b_ref, w1_ref, b1_ref, bng_ref, bnb_ref,
                w2_ref, b2_ref, out_ref, acc_pool, acc_cnt):
    n = pl.program_id(0)

    @pl.when(n == 0)
    def _():
        acc_pool[...] = jnp.zeros((NG, H), _f32)
        acc_cnt[...] = jnp.zeros((NG, H), _f32)

    hn = _layer_math(p0_ref[...], p1_ref[...], ms_ref[...], h_ref[...],
                     dis_ref[...], bc_ref[...], g_ref[...], b_ref[...])
    bb = batch_ref[0, 0, :]
    oh = (bb[None, :] == lax.broadcasted_iota(jnp.int32, (NG, BT), 0)).astype(_f32)
    acc_pool[...] = acc_pool[...] + _dot(oh, hn)
    cnt = jnp.sum(oh, axis=1, keepdims=True)
    acc_cnt[...] = acc_cnt[...] + jnp.broadcast_to(cnt, (NG, H))

    @pl.when(n == GRID - 1)
    def _():
        pooled = acc_pool[...] / jnp.maximum(acc_cnt[...], 1.0)
        z = _dot(pooled, w1_ref[...]) + b1_ref[...]
        z = z * bng_ref[...] * lax.rsqrt(jnp.float32(1.0 + 1e-5)) + bnb_ref[...]
        z = jax.nn.gelu(z)
        out_ref[...] = _dot(z, w2_ref[...]) + b2_ref[...]


def _tc_final(p0, p1, ms, h, dis, batch_r, bc_r, g_r, b_r,
              w1, b1_r, bng_r, bnb_r, w2, b2_r):
    blk = pl.BlockSpec((BT, H), lambda n: (n, 0))
    row = pl.BlockSpec((1, H), lambda n: (0, 0))
    mat = pl.BlockSpec((H, H), lambda n: (0, 0))
    return pl.pallas_call(
        _final_body,
        grid=(GRID,),
        in_specs=[blk, blk, blk, blk, blk,
                  pl.BlockSpec((1, 1, BT), lambda n: (n, 0, 0)),
                  row, row, row, mat, row, row, row, mat, row],
        out_specs=pl.BlockSpec((NG, H), lambda n: (0, 0)),
        out_shape=jax.ShapeDtypeStruct((NG, H), _f32),
        scratch_shapes=[
            pltpu.VMEM((NG, H), _f32),
            pltpu.VMEM((NG, H), _f32),
        ],
    )(p0, p1, ms, h, dis, batch_r, bc_r, g_r, b_r,
      w1, b1_r, bng_r, bnb_r, w2, b2_r)


def kernel(x, edge_index, edge_attr, batch, emb, Wc, bc, ln_g, ln_b,
           W1, b1, bn_g, bn_b, W2, b2):
    del edge_attr
    x = x.astype(jnp.int32)
    ei = edge_index.astype(jnp.int32)
    batch = batch.astype(jnp.int32)

    pad_e = E_PAD - E
    dummy_dst = N + (jnp.arange(pad_e, dtype=jnp.int32) % (NP - N))
    src_r = jnp.concatenate(
        [ei[0], jnp.zeros((pad_e,), jnp.int32)]).reshape(NW, NCH, CH)
    dst_r = jnp.concatenate([ei[1], dummy_dst]).reshape(NW, NCH, CH)

    x_r = jnp.pad(x, (0, NP - N), constant_values=V).reshape(GRID, 1, BT)
    batch_r = jnp.pad(batch, (0, NP - N),
                      constant_values=NG).reshape(GRID, 1, BT)
    emb_p = jnp.pad(emb, ((0, H - V), (0, 0)))

    ones_h = jnp.ones((CH, H), _f32)
    zeros_h = jnp.zeros((RPT, H), _f32)

    degp = _sc_deg_call(dst_r, ones_h, zeros_h)
    dis, h, ms = _tc_encoder(x_r, degp[0], degp[1], emb_p, Wc[0])

    for l in range(L - 1):
        part = _sc_scatter_call(ms, src_r, dst_r, zeros_h)
        h, ms = _tc_layer(part[0], part[1], ms, h, dis, Wc[l + 1],
                          bc[l].reshape(1, H), ln_g[l].reshape(1, H),
                          ln_b[l].reshape(1, H))

    part = _sc_scatter_call(ms, src_r, dst_r, zeros_h)
    out = _tc_final(part[0], part[1], ms, h, dis, batch_r,
                    bc[L - 1].reshape(1, H), ln_g[L - 1].reshape(1, H),
                    ln_b[L - 1].reshape(1, H),
                    W1, b1.reshape(1, H), bn_g.reshape(1, H),
                    bn_b.reshape(1, H), W2, b2.reshape(1, H))
    return out


# speedup vs baseline: 18.9332x; 2.9377x over previous
import functools

import jax
import jax.numpy as jnp
from jax import lax
from jax.experimental import pallas as pl
from jax.experimental.pallas import tpu as pltpu
from jax.experimental.pallas import tpu_sc as plsc

N = 10000
E = 320000
H = 128
NG = 256
L = 4
V = 119
NP = 10240
BT = 1024
GRID = NP // BT
NC = 2
NS = 16
NW = NC * NS
CH = 128
NCH = 80
EPW = NCH * CH
E_PAD = NW * EPW
RPT = NP // NS
NBLK = 40

_f32 = jnp.float32


def _sc_deg_call(dst_r, ones_h, zeros_h):
    mesh = plsc.VectorSubcoreMesh(core_axis_name="c", subcore_axis_name="s")

    @functools.partial(
        pl.kernel,
        mesh=mesh,
        out_type=jax.ShapeDtypeStruct((NC, NP, H), _f32),
        scratch_types=[
            pltpu.VMEM((NCH, CH), jnp.int32),
            pltpu.VMEM((CH, H), _f32),
            pltpu.VMEM_SHARED((NP, H), _f32),
        ],
    )
    def k(dst_hbm, ones_hbm, zeros_hbm, out_hbm, dst_v, ones_v, acc_sh):
        cid = lax.axis_index("c")
        sid = lax.axis_index("s")
        wid = cid * NS + sid
        pltpu.sync_copy(zeros_hbm, acc_sh.at[pl.ds(sid * RPT, RPT)])
        pltpu.sync_copy(ones_hbm, ones_v)
        pltpu.sync_copy(dst_hbm.at[wid], dst_v)
        plsc.subcore_barrier()

        def body(j, carry):
            pltpu.sync_copy(ones_v, acc_sh.at[dst_v.at[j]], add=True)
            return carry

        lax.fori_loop(0, NCH, body, 0)
        plsc.subcore_barrier()
        pltpu.sync_copy(acc_sh.at[pl.ds(sid * RPT, RPT)],
                        out_hbm.at[cid, pl.ds(sid * RPT, RPT)])

    return k(dst_r, ones_h, zeros_h)


def _sc_scatter_call(ms, src_r, dst_r, zeros_h):
    mesh = plsc.VectorSubcoreMesh(core_axis_name="c", subcore_axis_name="s")

    @functools.partial(
        pl.kernel,
        mesh=mesh,
        out_type=jax.ShapeDtypeStruct((NC, NP, H), _f32),
        scratch_types=[
            pltpu.VMEM((NBLK, CH), jnp.int32),
            pltpu.VMEM((NBLK, CH), jnp.int32),
            pltpu.VMEM((CH, H), _f32),
            pltpu.VMEM((CH, H), _f32),
            pltpu.VMEM_SHARED((NP, H), _f32),
            pltpu.SemaphoreType.DMA,
            pltpu.SemaphoreType.DMA,
        ],
    )
    def k(ms_hbm, src_hbm, dst_hbm, zeros_hbm, out_hbm,
          src_v, dst_v, buf0, buf1, acc_sh, g0, g1):
        cid = lax.axis_index("c")
        sid = lax.axis_index("s")
        wid = cid * NS + sid
        pltpu.sync_copy(zeros_hbm, acc_sh.at[pl.ds(sid * RPT, RPT)])
        plsc.subcore_barrier()

        bufs = (buf0, buf1)
        sems = (g0, g1)
        for hb in range(NCH // NBLK):
            pltpu.sync_copy(src_hbm.at[wid, pl.ds(hb * NBLK, NBLK)], src_v)
            pltpu.sync_copy(dst_hbm.at[wid, pl.ds(hb * NBLK, NBLK)], dst_v)
            pltpu.async_copy(ms_hbm.at[src_v.at[0]], buf0, g0)

            def body(i, carry):
                for b in range(2):
                    j = 2 * i + b
                    pltpu.make_async_copy(
                        ms_hbm.at[src_v.at[j]], bufs[b], sems[b]).wait()

                    @pl.when(j + 1 < NBLK)
                    def _():
                        pltpu.async_copy(ms_hbm.at[src_v.at[j + 1]],
                                         bufs[1 - b], sems[1 - b])

                    pltpu.sync_copy(bufs[b], acc_sh.at[dst_v.at[j]],
                                    add=True)
                return carry

            lax.fori_loop(0, NBLK // 2, body, 0)

        plsc.subcore_barrier()
        pltpu.sync_copy(acc_sh.at[pl.ds(sid * RPT, RPT)],
                        out_hbm.at[cid, pl.ds(sid * RPT, RPT)])

    return k(ms, src_r, dst_r, zeros_h)


def _dot(a, b):
    return jnp.dot(a, b, preferred_element_type=_f32)


def _encoder_body(x_ref, d0_ref, d1_ref, emb_ref, w0_ref,
                  dis_ref, h_ref, ms_ref):
    xb = x_ref[0, 0, :]
    oh = (xb[:, None] == lax.broadcasted_iota(jnp.int32, (BT, H), 1)).astype(_f32)
    h = _dot(oh, emb_ref[...])
    deg = 1.0 + d0_ref[:, 0:1] + d1_ref[:, 0:1]
    dis = lax.rsqrt(jnp.maximum(deg, 1.0))
    disb = jnp.broadcast_to(dis, (BT, H))
    dis_ref[...] = disb
    h_ref[...] = h
    ms_ref[...] = disb * _dot(h, w0_ref[...])


def _tc_encoder(x_r, d0, d1, emb_p, w0):
    return pl.pallas_call(
        _encoder_body,
        grid=(GRID,),
        in_specs=[
            pl.BlockSpec((1, 1, BT), lambda n: (n, 0, 0)),
            pl.BlockSpec((BT, H), lambda n: (n, 0)),
            pl.BlockSpec((BT, H), lambda n: (n, 0)),
            pl.BlockSpec((H, H), lambda n: (0, 0)),
            pl.BlockSpec((H, H), lambda n: (0, 0)),
        ],
        out_specs=[
            pl.BlockSpec((BT, H), lambda n: (n, 0)),
            pl.BlockSpec((BT, H), lambda n: (n, 0)),
            pl.BlockSpec((BT, H), lambda n: (n, 0)),
        ],
        out_shape=[
            jax.ShapeDtypeStruct((NP, H), _f32),
            jax.ShapeDtypeStruct((NP, H), _f32),
            jax.ShapeDtypeStruct((NP, H), _f32),
        ],
    )(x_r, d0, d1, emb_p, w0)


def _layer_math(p0, p1, ms, h, dis, bc_r, g_r, b_r):
    t = p0 + p1 + ms
    agg = dis * t + bc_r
    hh = jax.nn.gelu(agg) + h
    mu = jnp.mean(hh, axis=1, keepdims=True)
    var = jnp.mean((hh - mu) ** 2, axis=1, keepdims=True)
    return (hh - mu) * lax.rsqrt(var + 1e-5) * g_r + b_r


def _layer_body(p0_ref, p1_ref, ms_ref, h_ref, dis_ref, w_ref,
                bc_ref, g_ref, b_ref, hn_ref, msn_ref):
    hn = _layer_math(p0_ref[...], p1_ref[...], ms_ref[...], h_ref[...],
                     dis_ref[...], bc_ref[...], g_ref[...], b_ref[...])
    hn_ref[...] = hn
    msn_ref[...] = dis_ref[...] * _dot(hn, w_ref[...])


def _tc_layer(p0, p1, ms, h, dis, w_next, bc_r, g_r, b_r):
    blk = pl.BlockSpec((BT, H), lambda n: (n, 0))
    row = pl.BlockSpec((1, H), lambda n: (0, 0))
    return pl.pallas_call(
        _layer_body,
        grid=(GRID,),
        in_specs=[blk, blk, blk, blk, blk,
                  pl.BlockSpec((H, H), lambda n: (0, 0)), row, row, row],
        out_specs=[blk, blk],
        out_shape=[
            jax.ShapeDtypeStruct((NP, H), _f32),
            jax.ShapeDtypeStruct((NP, H), _f32),
        ],
    )(p0, p1, ms, h, dis, w_next, bc_r, g_r, b_r)


def _final_body(p0_ref, p1_ref, ms_ref, h_ref, dis_ref, batch_ref,
                bc_ref, g_ref, b_ref, w1_ref, b1_ref, bng_ref, bnb_ref,
                w2_ref, b2_ref, out_ref, acc_pool, acc_cnt):
    n = pl.program_id(0)

    @pl.when(n == 0)
    def _():
        acc_pool[...] = jnp.zeros((NG, H), _f32)
        acc_cnt[...] = jnp.zeros((NG, H), _f32)

    hn = _layer_math(p0_ref[...], p1_ref[...], ms_ref[...], h_ref[...],
                     dis_ref[...], bc_ref[...], g_ref[...], b_ref[...])
    bb = batch_ref[0, 0, :]
    oh = (bb[None, :] == lax.broadcasted_iota(jnp.int32, (NG, BT), 0)).astype(_f32)
    acc_pool[...] = acc_pool[...] + _dot(oh, hn)
    cnt = jnp.sum(oh, axis=1, keepdims=True)
    acc_cnt[...] = acc_cnt[...] + jnp.broadcast_to(cnt, (NG, H))

    @pl.when(n == GRID - 1)
    def _():
        pooled = acc_pool[...] / jnp.maximum(acc_cnt[...], 1.0)
        z = _dot(pooled, w1_ref[...]) + b1_ref[...]
        z = z * bng_ref[...] * lax.rsqrt(jnp.float32(1.0 + 1e-5)) + bnb_ref[...]
        z = jax.nn.gelu(z)
        out_ref[...] = _dot(z, w2_ref[...]) + b2_ref[...]


def _tc_final(p0, p1, ms, h, dis, batch_r, bc_r, g_r, b_r,
              w1, b1_r, bng_r, bnb_r, w2, b2_r):
    blk = pl.BlockSpec((BT, H), lambda n: (n, 0))
    row = pl.BlockSpec((1, H), lambda n: (0, 0))
    mat = pl.BlockSpec((H, H), lambda n: (0, 0))
    return pl.pallas_call(
        _final_body,
        grid=(GRID,),
        in_specs=[blk, blk, blk, blk, blk,
                  pl.BlockSpec((1, 1, BT), lambda n: (n, 0, 0)),
                  row, row, row, mat, row, row, row, mat, row],
        out_specs=pl.BlockSpec((NG, H), lambda n: (0, 0)),
        out_shape=jax.ShapeDtypeStruct((NG, H), _f32),
        scratch_shapes=[
            pltpu.VMEM((NG, H), _f32),
            pltpu.VMEM((NG, H), _f32),
        ],
    )(p0, p1, ms, h, dis, batch_r, bc_r, g_r, b_r,
      w1, b1_r, bng_r, bnb_r, w2, b2_r)


def kernel(x, edge_index, edge_attr, batch, emb, Wc, bc, ln_g, ln_b,
           W1, b1, bn_g, bn_b, W2, b2):
    del edge_attr
    x = x.astype(jnp.int32)
    ei = edge_index.astype(jnp.int32)
    batch = batch.astype(jnp.int32)

    pad_e = E_PAD - E
    dummy_dst = N + (jnp.arange(pad_e, dtype=jnp.int32) % (NP - N))
    dummy_src = jnp.arange(pad_e, dtype=jnp.int32) % N
    src_r = jnp.concatenate([ei[0], dummy_src]).reshape(NW, NCH, CH)
    dst_r = jnp.concatenate([ei[1], dummy_dst]).reshape(NW, NCH, CH)

    x_r = jnp.pad(x, (0, NP - N), constant_values=V).reshape(GRID, 1, BT)
    batch_r = jnp.pad(batch, (0, NP - N),
                      constant_values=NG).reshape(GRID, 1, BT)
    emb_p = jnp.pad(emb, ((0, H - V), (0, 0)))

    ones_h = jnp.ones((CH, H), _f32)
    zeros_h = jnp.zeros((RPT, H), _f32)

    degp = _sc_deg_call(dst_r, ones_h, zeros_h)
    dis, h, ms = _tc_encoder(x_r, degp[0], degp[1], emb_p, Wc[0])

    for l in range(L - 1):
        part = _sc_scatter_call(ms, src_r, dst_r, zeros_h)
        h, ms = _tc_layer(part[0], part[1], ms, h, dis, Wc[l + 1],
                          bc[l].reshape(1, H), ln_g[l].reshape(1, H),
                          ln_b[l].reshape(1, H))

    part = _sc_scatter_call(ms, src_r, dst_r, zeros_h)
    out = _tc_final(part[0], part[1], ms, h, dis, batch_r,
                    bc[L - 1].reshape(1, H), ln_g[L - 1].reshape(1, H),
                    ln_b[L - 1].reshape(1, H),
                    W1, b1.reshape(1, H), bn_g.reshape(1, H),
                    bn_b.reshape(1, H), W2, b2.reshape(1, H))
    return out


# speedup vs baseline: 18.9457x; 1.0007x over previous
import functools

import jax
import jax.numpy as jnp
from jax import lax
from jax.experimental import pallas as pl
from jax.experimental.pallas import tpu as pltpu
from jax.experimental.pallas import tpu_sc as plsc

N = 10000
E = 320000
H = 128
NG = 256
L = 4
V = 119
NP = 10240
BT = 1024
GRID = NP // BT
NC = 2
NS = 16
NW = NC * NS
CH = 128
NCH = 80
EPW = NCH * CH
E_PAD = NW * EPW
RPT = NP // NS
NBLK = 40

_f32 = jnp.float32


def _sc_deg_call(dst_r, ones_h, zeros_h):
    mesh = plsc.VectorSubcoreMesh(core_axis_name="c", subcore_axis_name="s")

    @functools.partial(
        pl.kernel,
        mesh=mesh,
        out_type=jax.ShapeDtypeStruct((NC, NP, H), _f32),
        scratch_types=[
            pltpu.VMEM((NCH, CH), jnp.int32),
            pltpu.VMEM((CH, H), _f32),
            pltpu.VMEM_SHARED((NP, H), _f32),
        ],
    )
    def k(dst_hbm, ones_hbm, zeros_hbm, out_hbm, dst_v, ones_v, acc_sh):
        cid = lax.axis_index("c")
        sid = lax.axis_index("s")
        wid = cid * NS + sid
        pltpu.sync_copy(zeros_hbm, acc_sh.at[pl.ds(sid * RPT, RPT)])
        pltpu.sync_copy(ones_hbm, ones_v)
        pltpu.sync_copy(dst_hbm.at[wid], dst_v)
        plsc.subcore_barrier()

        def body(j, carry):
            pltpu.sync_copy(ones_v, acc_sh.at[dst_v.at[j]], add=True)
            return carry

        lax.fori_loop(0, NCH, body, 0)
        plsc.subcore_barrier()
        pltpu.sync_copy(acc_sh.at[pl.ds(sid * RPT, RPT)],
                        out_hbm.at[cid, pl.ds(sid * RPT, RPT)])

    return k(dst_r, ones_h, zeros_h)


def _sc_scatter_call(ms, src_r, dst_r, zeros_h):
    mesh = plsc.VectorSubcoreMesh(core_axis_name="c", subcore_axis_name="s")

    @functools.partial(
        pl.kernel,
        mesh=mesh,
        out_type=jax.ShapeDtypeStruct((NC, NP, H), _f32),
        scratch_types=[
            pltpu.VMEM((NBLK, CH), jnp.int32),
            pltpu.VMEM((NBLK, CH), jnp.int32),
            pltpu.VMEM((CH, H), _f32),
            pltpu.VMEM((CH, H), _f32),
            pltpu.VMEM_SHARED((NP, H), _f32),
            pltpu.SemaphoreType.DMA,
            pltpu.SemaphoreType.DMA,
        ],
    )
    def k(ms_hbm, src_hbm, dst_hbm, zeros_hbm, out_hbm,
          src_v, dst_v, buf0, buf1, acc_sh, g0, g1):
        cid = lax.axis_index("c")
        sid = lax.axis_index("s")
        wid = cid * NS + sid
        pltpu.sync_copy(zeros_hbm, acc_sh.at[pl.ds(sid * RPT, RPT)])
        plsc.subcore_barrier()

        bufs = (buf0, buf1)
        sems = (g0, g1)
        for hb in range(NCH // NBLK):
            pltpu.sync_copy(src_hbm.at[wid, pl.ds(hb * NBLK, NBLK)], src_v)
            pltpu.sync_copy(dst_hbm.at[wid, pl.ds(hb * NBLK, NBLK)], dst_v)
            pltpu.async_copy(ms_hbm.at[src_v.at[0]], buf0, g0)

            def body(i, carry):
                for b in range(2):
                    j = 2 * i + b
                    pltpu.make_async_copy(
                        ms_hbm.at[src_v.at[j]], bufs[b], sems[b]).wait()

                    @pl.when(j + 1 < NBLK)
                    def _():
                        pltpu.async_copy(ms_hbm.at[src_v.at[j + 1]],
                                         bufs[1 - b], sems[1 - b])

                    pltpu.sync_copy(bufs[b], acc_sh.at[dst_v.at[j]],
                                    add=True)
                return carry

            lax.fori_loop(0, NBLK // 2, body, 0)

        plsc.subcore_barrier()
        pltpu.sync_copy(acc_sh.at[pl.ds(sid * RPT, RPT)],
                        out_hbm.at[cid, pl.ds(sid * RPT, RPT)])

    return k(ms, src_r, dst_r, zeros_h)


def _dot(a, b):
    return jnp.dot(a, b, preferred_element_type=_f32)


def _emb_body(x_ref, emb_ref, h_ref):
    xb = x_ref[0, 0, :]
    oh = (xb[:, None] == lax.broadcasted_iota(jnp.int32, (BT, H), 1)).astype(_f32)
    h_ref[...] = _dot(oh, emb_ref[...])


def _tc_emb(x_r, emb_p):
    return pl.pallas_call(
        _emb_body,
        grid=(GRID,),
        in_specs=[
            pl.BlockSpec((1, 1, BT), lambda n: (n, 0, 0)),
            pl.BlockSpec((H, H), lambda n: (0, 0)),
        ],
        out_specs=pl.BlockSpec((BT, H), lambda n: (n, 0)),
        out_shape=jax.ShapeDtypeStruct((NP, H), _f32),
    )(x_r, emb_p)


def _dis_ms_body(d0_ref, d1_ref, h_ref, w0_ref, dis_ref, ms_ref):
    deg = 1.0 + d0_ref[:, 0:1] + d1_ref[:, 0:1]
    dis = lax.rsqrt(jnp.maximum(deg, 1.0))
    disb = jnp.broadcast_to(dis, (BT, H))
    dis_ref[...] = disb
    ms_ref[...] = disb * _dot(h_ref[...], w0_ref[...])


def _tc_dis_ms(d0, d1, h, w0):
    blk = pl.BlockSpec((BT, H), lambda n: (n, 0))
    return pl.pallas_call(
        _dis_ms_body,
        grid=(GRID,),
        in_specs=[blk, blk, blk, pl.BlockSpec((H, H), lambda n: (0, 0))],
        out_specs=[blk, blk],
        out_shape=[
            jax.ShapeDtypeStruct((NP, H), _f32),
            jax.ShapeDtypeStruct((NP, H), _f32),
        ],
    )(d0, d1, h, w0)


def _layer_math(p0, p1, ms, h, dis, bc_r, g_r, b_r):
    t = p0 + p1 + ms
    agg = dis * t + bc_r
    hh = jax.nn.gelu(agg) + h
    mu = jnp.mean(hh, axis=1, keepdims=True)
    var = jnp.mean((hh - mu) ** 2, axis=1, keepdims=True)
    return (hh - mu) * lax.rsqrt(var + 1e-5) * g_r + b_r


def _layer_body(p0_ref, p1_ref, ms_ref, h_ref, dis_ref, w_ref,
                bc_ref, g_ref, b_ref, hn_ref, msn_ref):
    hn = _layer_math(p0_ref[...], p1_ref[...], ms_ref[...], h_ref[...],
                     dis_ref[...], bc_ref[...], g_ref[...], b_ref[...])
    hn_ref[...] = hn
    msn_ref[...] = dis_ref[...] * _dot(hn, w_ref[...])


def _tc_layer(p0, p1, ms, h, dis, w_next, bc_r, g_r, b_r):
    blk = pl.BlockSpec((BT, H), lambda n: (n, 0))
    row = pl.BlockSpec((1, H), lambda n: (0, 0))
    return pl.pallas_call(
        _layer_body,
        grid=(GRID,),
        in_specs=[blk, blk, blk, blk, blk,
                  pl.BlockSpec((H, H), lambda n: (0, 0)), row, row, row],
        out_specs=[blk, blk],
        out_shape=[
            jax.ShapeDtypeStruct((NP, H), _f32),
            jax.ShapeDtypeStruct((NP, H), _f32),
        ],
    )(p0, p1, ms, h, dis, w_next, bc_r, g_r, b_r)


def _final_body(p0_ref, p1_ref, ms_ref, h_ref, dis_ref, batch_ref,
                bc_ref, g_ref, b_ref, w1_ref, b1_ref, bng_ref, bnb_ref,
                w2_ref, b2_ref, out_ref, acc_pool, acc_cnt):
    n = pl.program_id(0)

    @pl.when(n == 0)
    def _():
        acc_pool[...] = jnp.zeros((NG, H), _f32)
        acc_cnt[...] = jnp.zeros((NG, H), _f32)

    hn = _layer_math(p0_ref[...], p1_ref[...], ms_ref[...], h_ref[...],
                     dis_ref[...], bc_ref[...], g_ref[...], b_ref[...])
    bb = batch_ref[0, 0, :]
    oh = (bb[None, :] == lax.broadcasted_iota(jnp.int32, (NG, BT), 0)).astype(_f32)
    acc_pool[...] = acc_pool[...] + _dot(oh, hn)
    cnt = jnp.sum(oh, axis=1, keepdims=True)
    acc_cnt[...] = acc_cnt[...] + jnp.broadcast_to(cnt, (NG, H))

    @pl.when(n == GRID - 1)
    def _():
        pooled = acc_pool[...] / jnp.maximum(acc_cnt[...], 1.0)
        z = _dot(pooled, w1_ref[...]) + b1_ref[...]
        z = z * bng_ref[...] * lax.rsqrt(jnp.float32(1.0 + 1e-5)) + bnb_ref[...]
        z = jax.nn.gelu(z)
        out_ref[...] = _dot(z, w2_ref[...]) + b2_ref[...]


def _tc_final(p0, p1, ms, h, dis, batch_r, bc_r, g_r, b_r,
              w1, b1_r, bng_r, bnb_r, w2, b2_r):
    blk = pl.BlockSpec((BT, H), lambda n: (n, 0))
    row = pl.BlockSpec((1, H), lambda n: (0, 0))
    mat = pl.BlockSpec((H, H), lambda n: (0, 0))
    return pl.pallas_call(
        _final_body,
        grid=(GRID,),
        in_specs=[blk, blk, blk, blk, blk,
                  pl.BlockSpec((1, 1, BT), lambda n: (n, 0, 0)),
                  row, row, row, mat, row, row, row, mat, row],
        out_specs=pl.BlockSpec((NG, H), lambda n: (0, 0)),
        out_shape=jax.ShapeDtypeStruct((NG, H), _f32),
        scratch_shapes=[
            pltpu.VMEM((NG, H), _f32),
            pltpu.VMEM((NG, H), _f32),
        ],
    )(p0, p1, ms, h, dis, batch_r, bc_r, g_r, b_r,
      w1, b1_r, bng_r, bnb_r, w2, b2_r)


def kernel(x, edge_index, edge_attr, batch, emb, Wc, bc, ln_g, ln_b,
           W1, b1, bn_g, bn_b, W2, b2):
    del edge_attr
    x = x.astype(jnp.int32)
    ei = edge_index.astype(jnp.int32)
    batch = batch.astype(jnp.int32)

    pad_e = E_PAD - E
    dummy_dst = N + (jnp.arange(pad_e, dtype=jnp.int32) % (NP - N))
    dummy_src = jnp.arange(pad_e, dtype=jnp.int32) % N
    src_r = jnp.concatenate([ei[0], dummy_src]).reshape(NW, NCH, CH)
    dst_r = jnp.concatenate([ei[1], dummy_dst]).reshape(NW, NCH, CH)

    x_r = jnp.pad(x, (0, NP - N), constant_values=V).reshape(GRID, 1, BT)
    batch_r = jnp.pad(batch, (0, NP - N),
                      constant_values=NG).reshape(GRID, 1, BT)
    emb_p = jnp.pad(emb, ((0, H - V), (0, 0)))

    ones_h = jnp.ones((CH, H), _f32)
    zeros_h = jnp.zeros((RPT, H), _f32)

    degp = _sc_deg_call(dst_r, ones_h, zeros_h)
    h = _tc_emb(x_r, emb_p)
    dis, ms = _tc_dis_ms(degp[0], degp[1], h, Wc[0])

    for l in range(L - 1):
        part = _sc_scatter_call(ms, src_r, dst_r, zeros_h)
        h, ms = _tc_layer(part[0], part[1], ms, h, dis, Wc[l + 1],
                          bc[l].reshape(1, H), ln_g[l].reshape(1, H),
                          ln_b[l].reshape(1, H))

    part = _sc_scatter_call(ms, src_r, dst_r, zeros_h)
    out = _tc_final(part[0], part[1], ms, h, dis, batch_r,
                    bc[L - 1].reshape(1, H), ln_g[L - 1].reshape(1, H),
                    ln_b[L - 1].reshape(1, H),
                    W1, b1.reshape(1, H), bn_g.reshape(1, H),
                    bn_b.reshape(1, H), W2, b2.reshape(1, H))
    return out
